# baseline (speedup 1.0000x reference)
import jax
import jax.numpy as jnp
from jax.experimental import pallas as pl
from jax.experimental.pallas import tpu as pltpu

_B, _DIN, _T = 8, 768, 1024
_K, _DC = 8192, 64
_TB = 256


def _vq_kernel(z_ref, ipv_ref, ipg_ref, ipb_ref, opv_ref, opg_ref, opb_ref,
               cb_ref, ze_ref, idx_ref, zqout_ref, loss_ref):
    t = pl.program_id(1)
    f32 = jnp.float32

    v = ipv_ref[...]
    nrm_in = jnp.sqrt(jnp.sum(v * v, axis=1, keepdims=True))
    w_in = ipg_ref[...] * v / jnp.clip(nrm_in, 1e-12)

    vo = opv_ref[...]
    nrm_out = jnp.sqrt(jnp.sum(vo * vo, axis=1, keepdims=True))
    w_out = opg_ref[...] * vo / jnp.clip(nrm_out, 1e-12)

    cb = cb_ref[...]
    cb_nrm = jnp.sqrt(jnp.sum(cb * cb, axis=1, keepdims=True))
    cbn = cb / jnp.clip(cb_nrm, 1e-12)
    s_col = jnp.sum(cbn * cbn, axis=1, keepdims=True)

    zb = z_ref[0]
    ze = jax.lax.dot_general(w_in.astype(jnp.bfloat16),
                             zb.astype(jnp.bfloat16),
                             (((1,), (0,)), ((), ())),
                             preferred_element_type=f32) + ipb_ref[...]
    ze_ref[0] = ze

    norm_e = jnp.sqrt(jnp.sum(ze * ze, axis=0, keepdims=True))
    en = ze / jnp.clip(norm_e, 1e-12)
    c_row = jnp.sum(en * en, axis=0, keepdims=True)
    dots = jax.lax.dot_general(cbn.astype(jnp.bfloat16),
                               en.astype(jnp.bfloat16),
                               (((1,), (0,)), ((), ())),
                               preferred_element_type=f32)
    dist = (c_row - 2.0 * dots) + s_col
    ch = 2048
    iota_ch = jax.lax.broadcasted_iota(jnp.int32, (ch, _TB), 0)
    acc_v = None
    for qi in range(_K // ch):
        dq = jax.lax.slice(dist, (qi * ch, 0), ((qi + 1) * ch, _TB))
        vq = jnp.min(dq, axis=0, keepdims=True)
        iq = jnp.min(jnp.where(dq == vq, iota_ch + qi * ch, _K), axis=0,
                     keepdims=True)
        if acc_v is None:
            acc_v = vq.astype(jnp.bfloat16).astype(f32)
            acc_i = iq
        else:
            keep = (acc_v < vq) | ((acc_v == vq) & (acc_i < iq))
            acc_v = jnp.where(keep, acc_v, vq).astype(jnp.bfloat16).astype(f32)
            acc_i = jnp.where(keep, acc_i, iq)
    idx_row = acc_i
    iota_k = jax.lax.broadcasted_iota(jnp.int32, dist.shape, 0)
    idx_ref[0] = jnp.broadcast_to(idx_row, (8, _TB))

    onehot = jnp.where(iota_k == idx_row, 1.0, 0.0).astype(f32)
    q = jax.lax.dot_general(cb, onehot, (((0,), (0,)), ((), ())),
                            preferred_element_type=f32,
                            precision=jax.lax.Precision.HIGHEST)

    diff = ze - q
    part = jnp.sum(diff * diff)

    @pl.when(t == 0)
    def _():
        loss_ref[...] = jnp.broadcast_to(part, (1, 8, 128))

    @pl.when(t != 0)
    def _():
        loss_ref[...] = loss_ref[...] + part

    norm_q = jnp.sqrt(jnp.sum(q * q, axis=0, keepdims=True))
    qn = q / jnp.clip(norm_q, 1e-12)
    r_un = en + qn
    r_nrm = jnp.sqrt(jnp.sum(r_un * r_un, axis=0, keepdims=True))
    r = r_un / jnp.clip(r_nrm, 1e-12)
    rde = jnp.sum(r * ze, axis=0, keepdims=True)
    ede = jnp.sum(en * ze, axis=0, keepdims=True)
    scaling = norm_q / jnp.clip(norm_e, 1e-8)
    rot = scaling * (ze - 2.0 * r * rde + 2.0 * qn * ede)

    out = jax.lax.dot_general(w_out, rot, (((1,), (0,)), ((), ())),
                              preferred_element_type=f32) + opb_ref[...]
    zqout_ref[0] = out


def kernel(z, in_proj_v, in_proj_g, in_proj_b, out_proj_v, out_proj_g,
           out_proj_b, codebook):
    B, DIN, T = z.shape
    K, DC = codebook.shape
    nt = T // _TB
    f32 = jnp.float32

    ipg = in_proj_g.reshape(DC, 1).astype(f32)
    ipb = in_proj_b.reshape(DC, 1).astype(f32)
    opg = out_proj_g.reshape(DIN, 1).astype(f32)
    opb = out_proj_b.reshape(DIN, 1).astype(f32)

    full = lambda shape: pl.BlockSpec(shape, lambda b, t: (0,) * len(shape))

    out_shapes = (
        jax.ShapeDtypeStruct((B, DC, T), f32),
        jax.ShapeDtypeStruct((B, 8, T), jnp.int32),
        jax.ShapeDtypeStruct((B, DIN, T), f32),
        jax.ShapeDtypeStruct((B, 8, 128), f32),
    )
    out_specs = (
        pl.BlockSpec((1, DC, _TB), lambda b, t: (b, 0, t)),
        pl.BlockSpec((1, 8, _TB), lambda b, t: (b, 0, t)),
        pl.BlockSpec((1, DIN, _TB), lambda b, t: (b, 0, t)),
        pl.BlockSpec((1, 8, 128), lambda b, t: (b, 0, 0)),
    )
    in_specs = [
        pl.BlockSpec((1, DIN, _TB), lambda b, t: (b, 0, t)),
        full((DC, DIN)),
        full((DC, 1)),
        full((DC, 1)),
        full((DIN, DC)),
        full((DIN, 1)),
        full((DIN, 1)),
        full((K, DC)),
    ]

    ze, idx, zqout, loss = pl.pallas_call(
        _vq_kernel,
        grid=(B, nt),
        in_specs=in_specs,
        out_specs=out_specs,
        out_shape=out_shapes,
        compiler_params=pltpu.CompilerParams(
            dimension_semantics=("arbitrary", "arbitrary")),
    )(z.astype(f32), in_proj_v.astype(f32), ipg, ipb,
      out_proj_v.astype(f32), opg, opb, codebook.astype(f32))

    indices = idx[:, 0, :]
    losses = loss[:, 0, 0] / (DC * T)
    return (zqout, losses, losses, indices, ze)


# speedup vs baseline: 1.6673x; 1.6673x over previous
import jax
import jax.numpy as jnp
from jax.experimental import pallas as pl
from jax.experimental.pallas import tpu as pltpu

_B, _DIN, _T = 8, 768, 1024
_K, _DC = 8192, 64
_TB = 256


def _vq_kernel(z_ref, ipv_ref, ipg_ref, ipb_ref, opv_ref, opg_ref, opb_ref,
               cb_ref, ze_ref, idx_ref, zqout_ref, loss_ref,
               w_in_ref, w_out_ref, cbn_ref, s_ref, cbhi_ref, cblo_ref):
    t = pl.program_id(1)
    b = pl.program_id(0)
    f32 = jnp.float32
    bf16 = jnp.bfloat16

    @pl.when(jnp.logical_and(b == 0, t == 0))
    def _prep():
        v = ipv_ref[...]
        nrm_in = jnp.sqrt(jnp.sum(v * v, axis=1, keepdims=True))
        w_in_ref[...] = (ipg_ref[...] * v / jnp.clip(nrm_in, 1e-12)).astype(bf16)

        vo = opv_ref[...]
        nrm_out = jnp.sqrt(jnp.sum(vo * vo, axis=1, keepdims=True))
        w_out_ref[...] = (opg_ref[...] * vo / jnp.clip(nrm_out, 1e-12)).astype(bf16)

        cb = cb_ref[...]
        cb_nrm = jnp.sqrt(jnp.sum(cb * cb, axis=1, keepdims=True))
        cbn = cb / jnp.clip(cb_nrm, 1e-12)
        cbn_ref[...] = cbn.astype(bf16)
        s_ref[...] = jnp.sum(cbn * cbn, axis=1, keepdims=True)
        hi = cb.astype(bf16)
        cbhi_ref[...] = hi
        cblo_ref[...] = (cb - hi.astype(f32)).astype(bf16)

    zb = z_ref[0]
    ze = jax.lax.dot_general(w_in_ref[...], zb.astype(bf16),
                             (((1,), (0,)), ((), ())),
                             preferred_element_type=f32) + ipb_ref[...]
    ze_ref[0] = ze

    norm_e = jnp.sqrt(jnp.sum(ze * ze, axis=0, keepdims=True))
    en = ze / jnp.clip(norm_e, 1e-12)
    c_row = jnp.sum(en * en, axis=0, keepdims=True)
    dots = jax.lax.dot_general(cbn_ref[...], en.astype(bf16),
                               (((1,), (0,)), ((), ())),
                               preferred_element_type=f32)
    dist = (c_row - 2.0 * dots) + s_ref[...]
    ch = 2048
    iota_ch = jax.lax.broadcasted_iota(jnp.int32, (ch, _TB), 0)
    acc_v = None
    for qi in range(_K // ch):
        dq = jax.lax.slice(dist, (qi * ch, 0), ((qi + 1) * ch, _TB))
        vq = jnp.min(dq, axis=0, keepdims=True)
        iq = jnp.min(jnp.where(dq == vq, iota_ch + qi * ch, _K), axis=0,
                     keepdims=True)
        if acc_v is None:
            acc_v = vq.astype(bf16).astype(f32)
            acc_i = iq
        else:
            keep = (acc_v < vq) | ((acc_v == vq) & (acc_i < iq))
            acc_v = jnp.where(keep, acc_v, vq).astype(bf16).astype(f32)
            acc_i = jnp.where(keep, acc_i, iq)
    idx_row = acc_i
    idx_ref[0] = jnp.broadcast_to(idx_row, (8, _TB))

    iota_k = jax.lax.broadcasted_iota(jnp.int32, (_K, _TB), 0)
    onehot = (iota_k == idx_row).astype(bf16)
    dn = (((0,), (0,)), ((), ()))
    q = (jax.lax.dot_general(cbhi_ref[...], onehot, dn, preferred_element_type=f32)
         + jax.lax.dot_general(cblo_ref[...], onehot, dn, preferred_element_type=f32))

    diff = ze - q
    part = jnp.sum(diff * diff)

    @pl.when(t == 0)
    def _():
        loss_ref[0] = jnp.broadcast_to(part, (8, 128))

    @pl.when(t != 0)
    def _():
        loss_ref[0] = loss_ref[0] + part

    norm_q = jnp.sqrt(jnp.sum(q * q, axis=0, keepdims=True))
    qn = q / jnp.clip(norm_q, 1e-12)
    r_un = en + qn
    r_nrm = jnp.sqrt(jnp.sum(r_un * r_un, axis=0, keepdims=True))
    r = r_un / jnp.clip(r_nrm, 1e-12)
    rde = jnp.sum(r * ze, axis=0, keepdims=True)
    ede = jnp.sum(en * ze, axis=0, keepdims=True)
    scaling = norm_q / jnp.clip(norm_e, 1e-8)
    rot = scaling * (ze - 2.0 * r * rde + 2.0 * qn * ede)

    out = jax.lax.dot_general(w_out_ref[...], rot.astype(bf16),
                              (((1,), (0,)), ((), ())),
                              preferred_element_type=f32) + opb_ref[...]
    zqout_ref[0] = out


def kernel(z, in_proj_v, in_proj_g, in_proj_b, out_proj_v, out_proj_g,
           out_proj_b, codebook):
    B, DIN, T = z.shape
    K, DC = codebook.shape
    nt = T // _TB
    f32 = jnp.float32

    ipg = in_proj_g.reshape(DC, 1).astype(f32)
    ipb = in_proj_b.reshape(DC, 1).astype(f32)
    opg = out_proj_g.reshape(DIN, 1).astype(f32)
    opb = out_proj_b.reshape(DIN, 1).astype(f32)

    full = lambda shape: pl.BlockSpec(shape, lambda b, t: (0,) * len(shape))

    out_shapes = (
        jax.ShapeDtypeStruct((B, DC, T), f32),
        jax.ShapeDtypeStruct((B, 8, T), jnp.int32),
        jax.ShapeDtypeStruct((B, DIN, T), f32),
        jax.ShapeDtypeStruct((B, 8, 128), f32),
    )
    out_specs = (
        pl.BlockSpec((1, DC, _TB), lambda b, t: (b, 0, t)),
        pl.BlockSpec((1, 8, _TB), lambda b, t: (b, 0, t)),
        pl.BlockSpec((1, DIN, _TB), lambda b, t: (b, 0, t)),
        pl.BlockSpec((1, 8, 128), lambda b, t: (b, 0, 0)),
    )
    in_specs = [
        pl.BlockSpec((1, DIN, _TB), lambda b, t: (b, 0, t)),
        full((DC, DIN)),
        full((DC, 1)),
        full((DC, 1)),
        full((DIN, DC)),
        full((DIN, 1)),
        full((DIN, 1)),
        full((K, DC)),
    ]
    scratch_shapes = [
        pltpu.VMEM((DC, DIN), jnp.bfloat16),
        pltpu.VMEM((DIN, DC), jnp.bfloat16),
        pltpu.VMEM((K, DC), jnp.bfloat16),
        pltpu.VMEM((K, 1), f32),
        pltpu.VMEM((K, DC), jnp.bfloat16),
        pltpu.VMEM((K, DC), jnp.bfloat16),
    ]

    ze, idx, zqout, loss = pl.pallas_call(
        _vq_kernel,
        grid=(B, nt),
        in_specs=in_specs,
        out_specs=out_specs,
        out_shape=out_shapes,
        scratch_shapes=scratch_shapes,
        compiler_params=pltpu.CompilerParams(
            dimension_semantics=("arbitrary", "arbitrary")),
    )(z.astype(f32), in_proj_v.astype(f32), ipg, ipb,
      out_proj_v.astype(f32), opg, opb, codebook.astype(f32))

    indices = idx[:, 0, :]
    losses = loss[:, 0, 0] / (DC * T)
    return (zqout, losses, losses, indices, ze)


# speedup vs baseline: 2.1962x; 1.3172x over previous
import functools

import jax
import jax.numpy as jnp
from jax import lax
from jax.experimental import pallas as pl
from jax.experimental.pallas import tpu as pltpu
from jax.experimental.pallas import tpu_sc as plsc

_B, _DIN, _T = 8, 768, 1024
_K, _DC = 8192, 64
_TB = 256
_NW = 32
_BPW = (_B * _T) // _NW


def _vq_a_kernel(z_ref, ipv_ref, ipg_ref, ipb_ref, cb_ref,
                 ze_ref, idx_ref, w_in_ref, cbn_ref, s_ref):
    t = pl.program_id(1)
    b = pl.program_id(0)
    f32 = jnp.float32
    bf16 = jnp.bfloat16

    @pl.when(jnp.logical_and(b == 0, t == 0))
    def _prep():
        v = ipv_ref[...]
        nrm_in = jnp.sqrt(jnp.sum(v * v, axis=1, keepdims=True))
        w_in_ref[...] = (ipg_ref[...] * v / jnp.clip(nrm_in, 1e-12)).astype(bf16)
        cb = cb_ref[...]
        cb_nrm = jnp.sqrt(jnp.sum(cb * cb, axis=1, keepdims=True))
        cbn = cb / jnp.clip(cb_nrm, 1e-12)
        cbn_ref[...] = cbn.astype(bf16)
        s_ref[...] = jnp.sum(cbn * cbn, axis=1, keepdims=True)

    zb = z_ref[0]
    ze = jax.lax.dot_general(w_in_ref[...], zb.astype(bf16),
                             (((1,), (0,)), ((), ())),
                             preferred_element_type=f32) + ipb_ref[...]
    ze_ref[0] = ze

    norm_e = jnp.sqrt(jnp.sum(ze * ze, axis=0, keepdims=True))
    en = ze / jnp.clip(norm_e, 1e-12)
    c_row = jnp.sum(en * en, axis=0, keepdims=True)
    dots = jax.lax.dot_general(cbn_ref[...], en.astype(bf16),
                               (((1,), (0,)), ((), ())),
                               preferred_element_type=f32)
    dist = (c_row - 2.0 * dots) + s_ref[...]
    ch = 2048
    iota_ch = jax.lax.broadcasted_iota(jnp.int32, (ch, _TB), 0)
    acc_v = None
    for qi in range(_K // ch):
        dq = jax.lax.slice(dist, (qi * ch, 0), ((qi + 1) * ch, _TB))
        vq = jnp.min(dq, axis=0, keepdims=True)
        iq = jnp.min(jnp.where(dq == vq, iota_ch + qi * ch, _K), axis=0,
                     keepdims=True)
        if acc_v is None:
            acc_v = vq.astype(bf16).astype(f32)
            acc_i = iq
        else:
            keep = (acc_v < vq) | ((acc_v == vq) & (acc_i < iq))
            acc_v = jnp.where(keep, acc_v, vq).astype(bf16).astype(f32)
            acc_i = jnp.where(keep, acc_i, iq)
    idx_ref[0] = jnp.broadcast_to(acc_i, (8, _TB))


_GD = 128
_GCH = 128


@functools.partial(
    pl.kernel,
    mesh=plsc.VectorSubcoreMesh(core_axis_name="c", subcore_axis_name="s"),
    out_type=jax.ShapeDtypeStruct((_B * _T, _GD), jnp.float32),
    scratch_types=[
        pltpu.VMEM((_GCH,), jnp.int32),
        pltpu.VMEM((_GCH, _GD), jnp.float32),
        pltpu.SemaphoreType.DMA,
    ],
)
def _sc_gather(table_hbm, idx_hbm, out_hbm, idx_v, rows_v, sem):
    wid = lax.axis_index("s") * 2 + lax.axis_index("c")
    base = wid * _BPW
    for ci in range(_BPW // _GCH):
        off = base + ci * _GCH
        pltpu.sync_copy(idx_hbm.at[pl.ds(off, _GCH)], idx_v)
        pltpu.async_copy(table_hbm.at[idx_v], rows_v, sem).wait()
        pltpu.sync_copy(rows_v, out_hbm.at[pl.ds(off, _GCH)])


def _vq_c_kernel(ze_ref, zq_ref, opv_ref, opg_ref, opb_ref,
                 zqout_ref, loss_ref, w_out_ref):
    t = pl.program_id(1)
    b = pl.program_id(0)
    f32 = jnp.float32
    bf16 = jnp.bfloat16

    @pl.when(jnp.logical_and(b == 0, t == 0))
    def _prep():
        vo = opv_ref[...]
        nrm_out = jnp.sqrt(jnp.sum(vo * vo, axis=1, keepdims=True))
        w_out_ref[...] = (opg_ref[...] * vo / jnp.clip(nrm_out, 1e-12)).astype(bf16)

    ze = ze_ref[0]
    q = jnp.transpose(zq_ref[...][:, :_DC])

    diff = ze - q
    part = jnp.sum(diff * diff)

    @pl.when(t == 0)
    def _():
        loss_ref[0] = jnp.broadcast_to(part, (8, 128))

    @pl.when(t != 0)
    def _():
        loss_ref[0] = loss_ref[0] + part

    norm_e = jnp.sqrt(jnp.sum(ze * ze, axis=0, keepdims=True))
    en = ze / jnp.clip(norm_e, 1e-12)
    norm_q = jnp.sqrt(jnp.sum(q * q, axis=0, keepdims=True))
    qn = q / jnp.clip(norm_q, 1e-12)
    r_un = en + qn
    r_nrm = jnp.sqrt(jnp.sum(r_un * r_un, axis=0, keepdims=True))
    r = r_un / jnp.clip(r_nrm, 1e-12)
    rde = jnp.sum(r * ze, axis=0, keepdims=True)
    ede = jnp.sum(en * ze, axis=0, keepdims=True)
    scaling = norm_q / jnp.clip(norm_e, 1e-8)
    rot = scaling * (ze - 2.0 * r * rde + 2.0 * qn * ede)

    out = jax.lax.dot_general(w_out_ref[...], rot.astype(bf16),
                              (((1,), (0,)), ((), ())),
                              preferred_element_type=f32) + opb_ref[...]
    zqout_ref[0] = out


def kernel(z, in_proj_v, in_proj_g, in_proj_b, out_proj_v, out_proj_g,
           out_proj_b, codebook):
    B, DIN, T = z.shape
    K, DC = codebook.shape
    nt = T // _TB
    f32 = jnp.float32

    ipg = in_proj_g.reshape(DC, 1).astype(f32)
    ipb = in_proj_b.reshape(DC, 1).astype(f32)
    opg = out_proj_g.reshape(DIN, 1).astype(f32)
    opb = out_proj_b.reshape(DIN, 1).astype(f32)
    cb32 = codebook.astype(f32)

    full = lambda shape: pl.BlockSpec(shape, lambda b, t: (0,) * len(shape))

    ze, idx = pl.pallas_call(
        _vq_a_kernel,
        grid=(B, nt),
        in_specs=[
            pl.BlockSpec((1, DIN, _TB), lambda b, t: (b, 0, t)),
            full((DC, DIN)),
            full((DC, 1)),
            full((DC, 1)),
            full((K, DC)),
        ],
        out_specs=(
            pl.BlockSpec((1, DC, _TB), lambda b, t: (b, 0, t)),
            pl.BlockSpec((1, 8, _TB), lambda b, t: (b, 0, t)),
        ),
        out_shape=(
            jax.ShapeDtypeStruct((B, DC, T), f32),
            jax.ShapeDtypeStruct((B, 8, T), jnp.int32),
        ),
        scratch_shapes=[
            pltpu.VMEM((DC, DIN), jnp.bfloat16),
            pltpu.VMEM((K, DC), jnp.bfloat16),
            pltpu.VMEM((K, 1), f32),
        ],
        compiler_params=pltpu.CompilerParams(
            dimension_semantics=("arbitrary", "arbitrary")),
    )(z.astype(f32), in_proj_v.astype(f32), ipg, ipb, cb32)

    indices = idx[:, 0, :]

    cb_pad = jnp.pad(cb32, ((0, 0), (0, _GD - DC)))
    zq_rows = _sc_gather(cb_pad, indices.reshape(B * T))

    zqout, loss = pl.pallas_call(
        _vq_c_kernel,
        grid=(B, nt),
        in_specs=[
            pl.BlockSpec((1, DC, _TB), lambda b, t: (b, 0, t)),
            pl.BlockSpec((_TB, _GD), lambda b, t: (b * (T // _TB) + t, 0)),
            full((DIN, DC)),
            full((DIN, 1)),
            full((DIN, 1)),
        ],
        out_specs=(
            pl.BlockSpec((1, DIN, _TB), lambda b, t: (b, 0, t)),
            pl.BlockSpec((1, 8, 128), lambda b, t: (b, 0, 0)),
        ),
        out_shape=(
            jax.ShapeDtypeStruct((B, DIN, T), f32),
            jax.ShapeDtypeStruct((B, 8, 128), f32),
        ),
        scratch_shapes=[
            pltpu.VMEM((DIN, DC), jnp.bfloat16),
        ],
        compiler_params=pltpu.CompilerParams(
            dimension_semantics=("arbitrary", "arbitrary")),
    )(ze, zq_rows, out_proj_v.astype(f32), opg, opb)

    losses = loss[:, 0, 0] / (DC * T)
    return (zqout, losses, losses, indices, ze)


# speedup vs baseline: 2.5878x; 1.1783x over previous
import functools

import jax
import jax.numpy as jnp
from jax import lax
from jax.experimental import pallas as pl
from jax.experimental.pallas import tpu as pltpu
from jax.experimental.pallas import tpu_sc as plsc

_B, _DIN, _T = 8, 768, 1024
_K, _DC = 8192, 64
_TB = 512
_NW = 32
_BPW = (_B * _T) // _NW


def _vq_a_kernel(z_ref, ipv_ref, ipg_ref, ipb_ref, cb_ref,
                 ze_ref, idx_ref, w_in_ref, cbn_ref, s_ref):
    t = pl.program_id(1)
    b = pl.program_id(0)
    f32 = jnp.float32
    bf16 = jnp.bfloat16

    @pl.when(jnp.logical_and(b == 0, t == 0))
    def _prep():
        v = ipv_ref[...]
        nrm_in = jnp.sqrt(jnp.sum(v * v, axis=1, keepdims=True))
        w_in_ref[...] = (ipg_ref[...] * v / jnp.clip(nrm_in, 1e-12)).astype(bf16)
        cb = cb_ref[...]
        cb_nrm = jnp.sqrt(jnp.sum(cb * cb, axis=1, keepdims=True))
        cbn = cb / jnp.clip(cb_nrm, 1e-12)
        cbn_ref[...] = cbn.astype(bf16)
        s_ref[...] = jnp.sum(cbn * cbn, axis=1, keepdims=True)

    zb = z_ref[0]
    ze = jax.lax.dot_general(w_in_ref[...], zb.astype(bf16),
                             (((1,), (0,)), ((), ())),
                             preferred_element_type=f32) + ipb_ref[...]
    ze_ref[0] = ze

    norm_e = jnp.sqrt(jnp.sum(ze * ze, axis=0, keepdims=True))
    en = ze / jnp.clip(norm_e, 1e-12)
    c_row = jnp.sum(en * en, axis=0, keepdims=True)
    dots = jax.lax.dot_general(cbn_ref[...], en.astype(bf16),
                               (((1,), (0,)), ((), ())),
                               preferred_element_type=f32)
    dist = (c_row - 2.0 * dots) + s_ref[...]
    ch = 2048
    iota_ch = jax.lax.broadcasted_iota(jnp.int32, (ch, _TB), 0)
    acc_v = None
    for qi in range(_K // ch):
        dq = jax.lax.slice(dist, (qi * ch, 0), ((qi + 1) * ch, _TB))
        vq = jnp.min(dq, axis=0, keepdims=True)
        iq = jnp.min(jnp.where(dq == vq, iota_ch + qi * ch, _K), axis=0,
                     keepdims=True)
        if acc_v is None:
            acc_v = vq.astype(bf16).astype(f32)
            acc_i = iq
        else:
            keep = (acc_v < vq) | ((acc_v == vq) & (acc_i < iq))
            acc_v = jnp.where(keep, acc_v, vq).astype(bf16).astype(f32)
            acc_i = jnp.where(keep, acc_i, iq)
    idx_ref[0] = jnp.broadcast_to(acc_i, (8, _TB))


_GD = 128
_GCH = 128


@functools.partial(
    pl.kernel,
    mesh=plsc.VectorSubcoreMesh(core_axis_name="c", subcore_axis_name="s"),
    out_type=jax.ShapeDtypeStruct((_B * _T, _GD), jnp.float32),
    scratch_types=[
        pltpu.VMEM((_GCH,), jnp.int32),
        pltpu.VMEM((_GCH, _GD), jnp.float32),
        pltpu.SemaphoreType.DMA,
    ],
)
def _sc_gather(table_hbm, idx_hbm, out_hbm, idx_v, rows_v, sem):
    wid = lax.axis_index("s") * 2 + lax.axis_index("c")
    base = wid * _BPW
    for ci in range(_BPW // _GCH):
        off = base + ci * _GCH
        pltpu.sync_copy(idx_hbm.at[pl.ds(off, _GCH)], idx_v)
        pltpu.async_copy(table_hbm.at[idx_v], rows_v, sem).wait()
        pltpu.sync_copy(rows_v, out_hbm.at[pl.ds(off, _GCH)])


def _vq_c_kernel(ze_ref, zq_ref, opv_ref, opg_ref, opb_ref,
                 zqout_ref, loss_ref, w_out_ref):
    t = pl.program_id(1)
    b = pl.program_id(0)
    f32 = jnp.float32
    bf16 = jnp.bfloat16

    @pl.when(jnp.logical_and(b == 0, t == 0))
    def _prep():
        vo = opv_ref[...]
        nrm_out = jnp.sqrt(jnp.sum(vo * vo, axis=1, keepdims=True))
        w_out_ref[...] = (opg_ref[...] * vo / jnp.clip(nrm_out, 1e-12)).astype(bf16)

    ze = ze_ref[0]
    q = jnp.transpose(zq_ref[...][:, :_DC])

    diff = ze - q
    part = jnp.sum(diff * diff)

    @pl.when(t == 0)
    def _():
        loss_ref[0] = jnp.broadcast_to(part, (8, 128))

    @pl.when(t != 0)
    def _():
        loss_ref[0] = loss_ref[0] + part

    norm_e = jnp.sqrt(jnp.sum(ze * ze, axis=0, keepdims=True))
    en = ze / jnp.clip(norm_e, 1e-12)
    norm_q = jnp.sqrt(jnp.sum(q * q, axis=0, keepdims=True))
    qn = q / jnp.clip(norm_q, 1e-12)
    r_un = en + qn
    r_nrm = jnp.sqrt(jnp.sum(r_un * r_un, axis=0, keepdims=True))
    r = r_un / jnp.clip(r_nrm, 1e-12)
    rde = jnp.sum(r * ze, axis=0, keepdims=True)
    ede = jnp.sum(en * ze, axis=0, keepdims=True)
    scaling = norm_q / jnp.clip(norm_e, 1e-8)
    rot = scaling * (ze - 2.0 * r * rde + 2.0 * qn * ede)

    out = jax.lax.dot_general(w_out_ref[...], rot.astype(bf16),
                              (((1,), (0,)), ((), ())),
                              preferred_element_type=f32) + opb_ref[...]
    zqout_ref[0] = out


def kernel(z, in_proj_v, in_proj_g, in_proj_b, out_proj_v, out_proj_g,
           out_proj_b, codebook):
    B, DIN, T = z.shape
    K, DC = codebook.shape
    nt = T // _TB
    f32 = jnp.float32

    ipg = in_proj_g.reshape(DC, 1).astype(f32)
    ipb = in_proj_b.reshape(DC, 1).astype(f32)
    opg = out_proj_g.reshape(DIN, 1).astype(f32)
    opb = out_proj_b.reshape(DIN, 1).astype(f32)
    cb32 = codebook.astype(f32)

    full = lambda shape: pl.BlockSpec(shape, lambda b, t: (0,) * len(shape))

    ze, idx = pl.pallas_call(
        _vq_a_kernel,
        grid=(B, nt),
        in_specs=[
            pl.BlockSpec((1, DIN, _TB), lambda b, t: (b, 0, t)),
            full((DC, DIN)),
            full((DC, 1)),
            full((DC, 1)),
            full((K, DC)),
        ],
        out_specs=(
            pl.BlockSpec((1, DC, _TB), lambda b, t: (b, 0, t)),
            pl.BlockSpec((1, 8, _TB), lambda b, t: (b, 0, t)),
        ),
        out_shape=(
            jax.ShapeDtypeStruct((B, DC, T), f32),
            jax.ShapeDtypeStruct((B, 8, T), jnp.int32),
        ),
        scratch_shapes=[
            pltpu.VMEM((DC, DIN), jnp.bfloat16),
            pltpu.VMEM((K, DC), jnp.bfloat16),
            pltpu.VMEM((K, 1), f32),
        ],
        compiler_params=pltpu.CompilerParams(
            dimension_semantics=("arbitrary", "arbitrary")),
    )(z.astype(f32), in_proj_v.astype(f32), ipg, ipb, cb32)

    indices = idx[:, 0, :]

    cb_pad = jnp.pad(cb32, ((0, 0), (0, _GD - DC)))
    zq_rows = _sc_gather(cb_pad, indices.reshape(B * T))

    zqout, loss = pl.pallas_call(
        _vq_c_kernel,
        grid=(B, nt),
        in_specs=[
            pl.BlockSpec((1, DC, _TB), lambda b, t: (b, 0, t)),
            pl.BlockSpec((_TB, _GD), lambda b, t: (b * (T // _TB) + t, 0)),
            full((DIN, DC)),
            full((DIN, 1)),
            full((DIN, 1)),
        ],
        out_specs=(
            pl.BlockSpec((1, DIN, _TB), lambda b, t: (b, 0, t)),
            pl.BlockSpec((1, 8, 128), lambda b, t: (b, 0, 0)),
        ),
        out_shape=(
            jax.ShapeDtypeStruct((B, DIN, T), f32),
            jax.ShapeDtypeStruct((B, 8, 128), f32),
        ),
        scratch_shapes=[
            pltpu.VMEM((DIN, DC), jnp.bfloat16),
        ],
        compiler_params=pltpu.CompilerParams(
            dimension_semantics=("arbitrary", "arbitrary")),
    )(ze, zq_rows, out_proj_v.astype(f32), opg, opb)

    losses = loss[:, 0, 0] / (DC * T)
    return (zqout, losses, losses, indices, ze)


# speedup vs baseline: 2.7577x; 1.0657x over previous
import functools

import jax
import jax.numpy as jnp
from jax import lax
from jax.experimental import pallas as pl
from jax.experimental.pallas import tpu as pltpu
from jax.experimental.pallas import tpu_sc as plsc

_B, _DIN, _T = 8, 768, 1024
_K, _DC = 8192, 64
_TB = 1024
_NW = 32
_BPW = (_B * _T) // _NW


def _vq_a_kernel(z_ref, ipv_ref, ipg_ref, ipb_ref, cb_ref,
                 ze_ref, idx_ref, w_in_ref, cbn_ref, s_ref):
    t = pl.program_id(1)
    b = pl.program_id(0)
    f32 = jnp.float32
    bf16 = jnp.bfloat16

    @pl.when(jnp.logical_and(b == 0, t == 0))
    def _prep():
        v = ipv_ref[...]
        nrm_in = jnp.sqrt(jnp.sum(v * v, axis=1, keepdims=True))
        w_in_ref[...] = (ipg_ref[...] * v / jnp.clip(nrm_in, 1e-12)).astype(bf16)
        cb = cb_ref[...]
        cb_nrm = jnp.sqrt(jnp.sum(cb * cb, axis=1, keepdims=True))
        cbn = cb / jnp.clip(cb_nrm, 1e-12)
        cbn_ref[...] = cbn.astype(bf16)
        s_ref[...] = jnp.sum(cbn * cbn, axis=1, keepdims=True)

    zb = z_ref[0]
    ze = jax.lax.dot_general(w_in_ref[...], zb.astype(bf16),
                             (((1,), (0,)), ((), ())),
                             preferred_element_type=f32) + ipb_ref[...]
    ze_ref[0] = ze

    norm_e = jnp.sqrt(jnp.sum(ze * ze, axis=0, keepdims=True))
    en = ze / jnp.clip(norm_e, 1e-12)
    c_row = jnp.sum(en * en, axis=0, keepdims=True)
    dots = jax.lax.dot_general(cbn_ref[...], en.astype(bf16),
                               (((1,), (0,)), ((), ())),
                               preferred_element_type=f32)
    dist = (c_row - 2.0 * dots) + s_ref[...]
    ch = 2048
    iota_ch = jax.lax.broadcasted_iota(jnp.int32, (ch, _TB), 0)
    acc_v = None
    for qi in range(_K // ch):
        dq = jax.lax.slice(dist, (qi * ch, 0), ((qi + 1) * ch, _TB))
        vq = jnp.min(dq, axis=0, keepdims=True)
        iq = jnp.min(jnp.where(dq == vq, iota_ch + qi * ch, _K), axis=0,
                     keepdims=True)
        if acc_v is None:
            acc_v = vq.astype(bf16).astype(f32)
            acc_i = iq
        else:
            keep = (acc_v < vq) | ((acc_v == vq) & (acc_i < iq))
            acc_v = jnp.where(keep, acc_v, vq).astype(bf16).astype(f32)
            acc_i = jnp.where(keep, acc_i, iq)
    idx_ref[0] = jnp.broadcast_to(acc_i, (8, _TB))


_GD = 128
_GCH = 128


@functools.partial(
    pl.kernel,
    mesh=plsc.VectorSubcoreMesh(core_axis_name="c", subcore_axis_name="s"),
    out_type=jax.ShapeDtypeStruct((_B * _T, _GD), jnp.float32),
    scratch_types=[
        pltpu.VMEM((_GCH,), jnp.int32),
        pltpu.VMEM((_GCH, _GD), jnp.float32),
        pltpu.SemaphoreType.DMA,
    ],
)
def _sc_gather(table_hbm, idx_hbm, out_hbm, idx_v, rows_v, sem):
    wid = lax.axis_index("s") * 2 + lax.axis_index("c")
    base = wid * _BPW
    for ci in range(_BPW // _GCH):
        off = base + ci * _GCH
        pltpu.sync_copy(idx_hbm.at[pl.ds(off, _GCH)], idx_v)
        pltpu.async_copy(table_hbm.at[idx_v], rows_v, sem).wait()
        pltpu.sync_copy(rows_v, out_hbm.at[pl.ds(off, _GCH)])


def _vq_c_kernel(ze_ref, zq_ref, opv_ref, opg_ref, opb_ref,
                 zqout_ref, loss_ref, w_out_ref):
    t = pl.program_id(1)
    b = pl.program_id(0)
    f32 = jnp.float32
    bf16 = jnp.bfloat16

    @pl.when(jnp.logical_and(b == 0, t == 0))
    def _prep():
        vo = opv_ref[...]
        nrm_out = jnp.sqrt(jnp.sum(vo * vo, axis=1, keepdims=True))
        w_out_ref[...] = (opg_ref[...] * vo / jnp.clip(nrm_out, 1e-12)).astype(bf16)

    ze = ze_ref[0]
    q = jnp.transpose(zq_ref[...][:, :_DC])

    diff = ze - q
    part = jnp.sum(diff * diff)

    @pl.when(t == 0)
    def _():
        loss_ref[0] = jnp.broadcast_to(part, (8, 128))

    @pl.when(t != 0)
    def _():
        loss_ref[0] = loss_ref[0] + part

    norm_e = jnp.sqrt(jnp.sum(ze * ze, axis=0, keepdims=True))
    en = ze / jnp.clip(norm_e, 1e-12)
    norm_q = jnp.sqrt(jnp.sum(q * q, axis=0, keepdims=True))
    qn = q / jnp.clip(norm_q, 1e-12)
    r_un = en + qn
    r_nrm = jnp.sqrt(jnp.sum(r_un * r_un, axis=0, keepdims=True))
    r = r_un / jnp.clip(r_nrm, 1e-12)
    rde = jnp.sum(r * ze, axis=0, keepdims=True)
    ede = jnp.sum(en * ze, axis=0, keepdims=True)
    scaling = norm_q / jnp.clip(norm_e, 1e-8)
    rot = scaling * (ze - 2.0 * r * rde + 2.0 * qn * ede)

    out = jax.lax.dot_general(w_out_ref[...], rot.astype(bf16),
                              (((1,), (0,)), ((), ())),
                              preferred_element_type=f32) + opb_ref[...]
    zqout_ref[0] = out


def kernel(z, in_proj_v, in_proj_g, in_proj_b, out_proj_v, out_proj_g,
           out_proj_b, codebook):
    B, DIN, T = z.shape
    K, DC = codebook.shape
    nt = T // _TB
    f32 = jnp.float32

    ipg = in_proj_g.reshape(DC, 1).astype(f32)
    ipb = in_proj_b.reshape(DC, 1).astype(f32)
    opg = out_proj_g.reshape(DIN, 1).astype(f32)
    opb = out_proj_b.reshape(DIN, 1).astype(f32)
    cb32 = codebook.astype(f32)

    full = lambda shape: pl.BlockSpec(shape, lambda b, t: (0,) * len(shape))

    ze, idx = pl.pallas_call(
        _vq_a_kernel,
        grid=(B, nt),
        in_specs=[
            pl.BlockSpec((1, DIN, _TB), lambda b, t: (b, 0, t)),
            full((DC, DIN)),
            full((DC, 1)),
            full((DC, 1)),
            full((K, DC)),
        ],
        out_specs=(
            pl.BlockSpec((1, DC, _TB), lambda b, t: (b, 0, t)),
            pl.BlockSpec((1, 8, _TB), lambda b, t: (b, 0, t)),
        ),
        out_shape=(
            jax.ShapeDtypeStruct((B, DC, T), f32),
            jax.ShapeDtypeStruct((B, 8, T), jnp.int32),
        ),
        scratch_shapes=[
            pltpu.VMEM((DC, DIN), jnp.bfloat16),
            pltpu.VMEM((K, DC), jnp.bfloat16),
            pltpu.VMEM((K, 1), f32),
        ],
        compiler_params=pltpu.CompilerParams(
            dimension_semantics=("arbitrary", "arbitrary")),
    )(z.astype(f32), in_proj_v.astype(f32), ipg, ipb, cb32)

    indices = idx[:, 0, :]

    cb_pad = jnp.pad(cb32, ((0, 0), (0, _GD - DC)))
    zq_rows = _sc_gather(cb_pad, indices.reshape(B * T))

    zqout, loss = pl.pallas_call(
        _vq_c_kernel,
        grid=(B, nt),
        in_specs=[
            pl.BlockSpec((1, DC, _TB), lambda b, t: (b, 0, t)),
            pl.BlockSpec((_TB, _GD), lambda b, t: (b * (T // _TB) + t, 0)),
            full((DIN, DC)),
            full((DIN, 1)),
            full((DIN, 1)),
        ],
        out_specs=(
            pl.BlockSpec((1, DIN, _TB), lambda b, t: (b, 0, t)),
            pl.BlockSpec((1, 8, 128), lambda b, t: (b, 0, 0)),
        ),
        out_shape=(
            jax.ShapeDtypeStruct((B, DIN, T), f32),
            jax.ShapeDtypeStruct((B, 8, 128), f32),
        ),
        scratch_shapes=[
            pltpu.VMEM((DIN, DC), jnp.bfloat16),
        ],
        compiler_params=pltpu.CompilerParams(
            dimension_semantics=("arbitrary", "arbitrary")),
    )(ze, zq_rows, out_proj_v.astype(f32), opg, opb)

    losses = loss[:, 0, 0] / (DC * T)
    return (zqout, losses, losses, indices, ze)
